# Initial kernel scaffold; baseline (speedup 1.0000x reference)
import jax
import jax.numpy as jnp
from jax.experimental import pallas as pl


def kernel(h, h_hat, edge_index, edge_index_hat, W1, b1, W2, b2):
    raise NotImplementedError("write your pallas kernel here")


# speedup vs baseline: 2.6177x; 2.6177x over previous
import jax
import jax.numpy as jnp
from jax import lax
from jax.experimental import pallas as pl
from jax.experimental.pallas import tpu as pltpu
from jax.experimental.pallas import tpu_sc as plsc

NN = 10000
DD = 128
DE = 128
EE = 320000
NPAD = 10240
EPAD = 327680
CHUNK = 128
NCHUNKS = EPAD // CHUNK
NTILES = 16
CPT = NCHUNKS // NTILES
GCH = 16
NGROUPS = CPT // GCH
RPT = NPAD // NTILES

LMIX = 0.5
GAMMA = 1.0
ALPHA = 0.5
BETA = 0.5

BR = 400
BC = 1000
BRC = 1000


def _make_seg_sum(width, do_gather=True):
    mesh = plsc.VectorSubcoreMesh(core_axis_name="c", subcore_axis_name="s")
    out_type = [jax.ShapeDtypeStruct((2 * NPAD, width), jnp.float32)]
    scratch = [
        pltpu.VMEM((GCH, CHUNK), jnp.int32),
        pltpu.VMEM((GCH, CHUNK), jnp.int32),
        pltpu.VMEM((CHUNK, width), jnp.float32),
        pltpu.VMEM_SHARED((NPAD, width), jnp.float32),
        pltpu.SemaphoreType.DMA,
    ]

    def body(tab, src, dst, zf, s_out, idxs_v, idxd_v, rows_v, acc_sh, sem):
        c = lax.axis_index("c")
        s = lax.axis_index("s")
        row0 = s * RPT
        pltpu.sync_copy(zf, rows_v)

        def zero_body(k, carry):
            pltpu.sync_copy(rows_v, acc_sh.at[pl.ds(row0 + k * CHUNK,
                                                    CHUNK)])
            return carry

        lax.fori_loop(0, RPT // CHUNK, zero_body, 0)
        chunk0 = c * NCHUNKS + s * CPT
        plsc.subcore_barrier()
        if not do_gather:
            pltpu.sync_copy(tab.at[pl.ds(0, CHUNK)], rows_v)

        def group_body(g, carry):
            g0 = chunk0 + g * GCH
            if do_gather:
                pltpu.sync_copy(src.at[pl.ds(g0, GCH)], idxs_v)
            pltpu.sync_copy(dst.at[pl.ds(g0, GCH)], idxd_v)

            def chunk_body(j, carry2):
                if do_gather:
                    pltpu.async_copy(tab.at[idxs_v.at[j]], rows_v,
                                     sem).wait()
                pltpu.sync_copy(rows_v, acc_sh.at[idxd_v.at[j]], add=True)
                return carry2

            return lax.fori_loop(0, GCH, chunk_body, carry)

        lax.fori_loop(0, NGROUPS, group_body, 0)
        plsc.subcore_barrier()
        out_row0 = c * NPAD + row0

        def wb_body(k, carry):
            r0 = row0 + k * CHUNK
            pltpu.sync_copy(acc_sh.at[pl.ds(r0, CHUNK)], rows_v)
            pltpu.sync_copy(rows_v, s_out.at[pl.ds(out_row0 + k * CHUNK,
                                                   CHUNK)])
            return carry

        lax.fori_loop(0, RPT // CHUNK, wb_body, 0)

    return pl.kernel(body, mesh=mesh, out_type=out_type,
                     scratch_types=scratch)


_seg_sum_cache = {}


def _get_seg_sum(width, do_gather=True):
    key = (width, do_gather)
    if key not in _seg_sum_cache:
        _seg_sum_cache[key] = _make_seg_sum(width, do_gather)
    return _seg_sum_cache[key]


def _encoder_body(x_ref, w_ref, b_ref, o_ref):
    y = jnp.dot(x_ref[0], w_ref[0], preferred_element_type=jnp.float32)
    y = jnp.maximum(y + b_ref[0], 0.0)
    nrm = jnp.sqrt(jnp.sum(y * y, axis=1, keepdims=True))
    o_ref[0] = y / jnp.maximum(nrm, 1e-12)


def _combine1_body(s_ref, d_ref, o_ref):
    deg = jnp.maximum(d_ref[0][:, :1], 1.0)
    o_ref[0] = s_ref[0] / deg


def _combine2_body(n1_ref, s2_ref, d_ref, o_ref):
    deg = jnp.maximum(d_ref[0][:, :1], 1.0)
    o_ref[0] = LMIX * n1_ref[0] + (1.0 - LMIX) * (s2_ref[0] / deg)


def _denom_body(a_ref, b0_ref, b1_ref, o_ref):
    j = pl.program_id(1)
    a = a_ref[0]
    dn = (((1,), (1,)), ((), ()))
    s0 = jnp.exp(lax.dot_general(a, b0_ref[0], dn,
                                 preferred_element_type=jnp.float32))
    s1 = jnp.exp(lax.dot_general(a, b1_ref[0], dn,
                                 preferred_element_type=jnp.float32))
    part = (jnp.sum(s0, axis=1, keepdims=True)
            + jnp.sum(s1, axis=1, keepdims=True))

    @pl.when(j == 0)
    def _():
        o_ref[...] = part

    @pl.when(j != 0)
    def _():
        o_ref[...] += part


def _softplus(x):
    return jnp.maximum(x, 0.0) + jnp.log(1.0 + jnp.exp(-jnp.abs(x)))


def _final_body(he_ref, hep_ref, me_ref, mep_ref, den_ref, o_ref):
    outs = []
    for c in range(2):
        hec = he_ref[c]
        mec = me_ref[c]
        hprev = jnp.concatenate([hep_ref[c, BR - 1:, :], hec[:-1]], axis=0)
        mprev = jnp.concatenate([mep_ref[c, BR - 1:, :], mec[:-1]], axis=0)
        pos = jnp.sum(hec * mec, axis=1)
        neg = jnp.sum(hec * mprev, axis=1)
        negn = jnp.sum(hec * hprev, axis=1)
        outs.append(jnp.sum(_softplus(-pos)))
        outs.append(jnp.sum(_softplus(neg)))
        outs.append(jnp.sum(_softplus(negn)))
    posscore = jnp.sum(me_ref[0] * me_ref[1], axis=1)
    outs.append(jnp.sum(jnp.log(den_ref[:, 0]) - posscore))
    outs.append(outs[0] * 0.0)
    for k, v in enumerate(outs):
        o_ref[0, k, :] = jnp.full((DD,), 0.0, jnp.float32) + v


def kernel(h, h_hat, edge_index, edge_index_hat, W1, b1, W2, b2):
    f32 = jnp.float32
    ni = NN // BR

    hs = jnp.stack([h, h_hat])
    Ws = jnp.stack([W1, W2])
    bs = jnp.stack([b1, b2]).reshape(2, 1, DD)

    pad = EPAD - EE
    pad_src = jnp.zeros((pad,), jnp.int32)
    pad_dst = jnp.full((pad,), NN, jnp.int32)
    src0 = jnp.concatenate([edge_index[0], pad_src])
    dst0 = jnp.concatenate([edge_index[1], pad_dst])
    src1 = jnp.concatenate([edge_index_hat[0] + NN, pad_src + NN])
    dst1 = jnp.concatenate([edge_index_hat[1], pad_dst])
    src_all = jnp.concatenate([src0, src1]).reshape(2 * NCHUNKS, CHUNK)
    dst_all = jnp.concatenate([dst0, dst1]).reshape(2 * NCHUNKS, CHUNK)

    zf = jnp.zeros((CHUNK, DD), f32)
    ones_tab = jnp.ones((CHUNK, DD), f32)

    he = pl.pallas_call(
        _encoder_body,
        grid=(2, ni),
        in_specs=[pl.BlockSpec((1, BR, DD), lambda c, i: (c, i, 0)),
                  pl.BlockSpec((1, DD, DD), lambda c, i: (c, 0, 0)),
                  pl.BlockSpec((1, 1, DD), lambda c, i: (c, 0, 0))],
        out_specs=pl.BlockSpec((1, BR, DD), lambda c, i: (c, i, 0)),
        out_shape=jax.ShapeDtypeStruct((2, NN, DD), f32),
    )(hs, Ws, bs)

    (degs,) = _get_seg_sum(DD, do_gather=False)(ones_tab, src_all, dst_all,
                                                zf)
    degs = degs.reshape(2, NPAD, DD)

    (s1,) = _get_seg_sum(DD)(he.reshape(2 * NN, DD), src_all, dst_all, zf)
    s1 = s1.reshape(2, NPAD, DD)

    neigh1 = pl.pallas_call(
        _combine1_body,
        grid=(2, ni),
        in_specs=[pl.BlockSpec((1, BR, DD), lambda c, i: (c, i, 0)),
                  pl.BlockSpec((1, BR, DD), lambda c, i: (c, i, 0))],
        out_specs=pl.BlockSpec((1, BR, DD), lambda c, i: (c, i, 0)),
        out_shape=jax.ShapeDtypeStruct((2, NN, DD), f32),
    )(s1, degs)

    (s2,) = _get_seg_sum(DD)(neigh1.reshape(2 * NN, DD), src_all, dst_all,
                             zf)
    s2 = s2.reshape(2, NPAD, DD)

    mean = pl.pallas_call(
        _combine2_body,
        grid=(2, ni),
        in_specs=[pl.BlockSpec((1, BR, DD), lambda c, i: (c, i, 0)),
                  pl.BlockSpec((1, BR, DD), lambda c, i: (c, i, 0)),
                  pl.BlockSpec((1, BR, DD), lambda c, i: (c, i, 0))],
        out_specs=pl.BlockSpec((1, BR, DD), lambda c, i: (c, i, 0)),
        out_shape=jax.ShapeDtypeStruct((2, NN, DD), f32),
    )(neigh1, s2, degs)

    den = pl.pallas_call(
        _denom_body,
        grid=(NN // BRC, NN // BC),
        in_specs=[pl.BlockSpec((1, BRC, DD), lambda i, j: (0, i, 0)),
                  pl.BlockSpec((1, BC, DD), lambda i, j: (0, j, 0)),
                  pl.BlockSpec((1, BC, DD), lambda i, j: (1, j, 0))],
        out_specs=pl.BlockSpec((BRC, 1), lambda i, j: (i, 0)),
        out_shape=jax.ShapeDtypeStruct((NN, 1), f32),
    )(mean, mean, mean)

    prev = lambda i: (i + ni - 1) % ni
    parts = pl.pallas_call(
        _final_body,
        grid=(ni,),
        in_specs=[pl.BlockSpec((2, BR, DD), lambda i: (0, i, 0)),
                  pl.BlockSpec((2, BR, DD), lambda i: (0, prev(i), 0)),
                  pl.BlockSpec((2, BR, DD), lambda i: (0, i, 0)),
                  pl.BlockSpec((2, BR, DD), lambda i: (0, prev(i), 0)),
                  pl.BlockSpec((BR, 1), lambda i: (i, 0))],
        out_specs=pl.BlockSpec((1, 8, DD), lambda i: (i, 0, 0)),
        out_shape=jax.ShapeDtypeStruct((ni, 8, DD), f32),
    )(he, he, mean, mean, den)

    S = jnp.sum(parts[:, :, 0], axis=0)
    nf = float(NN)
    l1, l2, l2n = S[0] / nf, S[1] / nf, S[2] / nf
    l1h, l2h, l2nh = S[3] / nf, S[4] / nf, S[5] / nf
    loss_sup = S[6] / (2.0 * nf)
    l_ns = ALPHA * (l1 + l2) + (1.0 - ALPHA) * (l1h + l2h)
    l_nn = ALPHA * (l1 + l2n) + (1.0 - ALPHA) * (l1h + l2nh)
    total = (BETA * l_ns + (1.0 - BETA) * l_nn + GAMMA * loss_sup) \
        / ((GAMMA + 1.0) / 2.0)
    return (total, l1, l2)
